# Initial kernel scaffold; baseline (speedup 1.0000x reference)
import jax
import jax.numpy as jnp
from jax.experimental import pallas as pl


def kernel(feature_tensor, shape_ids, word_ids, shape_table, word_table):
    raise NotImplementedError("write your pallas kernel here")


# speedup vs baseline: 1.7661x; 1.7661x over previous
import functools

import jax
import jax.numpy as jnp
from jax import lax
from jax.experimental import pallas as pl
from jax.experimental.pallas import tpu as pltpu
from jax.experimental.pallas import tpu_sc as plsc

B, S, F = 1024, 200, 64
SD, WD = 32, 32
OUT_D = F + SD + WD
N = B * S
NUM_CORES = 2
NUM_SUBCORES = 16
NW = NUM_CORES * NUM_SUBCORES
TOK_W = N // NW
C = 128
ITERS = TOK_W // C

_mesh = plsc.VectorSubcoreMesh(core_axis_name="c", subcore_axis_name="s")


@functools.partial(
    pl.kernel,
    mesh=_mesh,
    compiler_params=pltpu.CompilerParams(use_tc_tiling_on_sc=False),
    out_type=jax.ShapeDtypeStruct((N, OUT_D), jnp.float32),
    scratch_types=[
        pltpu.VMEM((C,), jnp.int32),
        pltpu.VMEM((C,), jnp.int32),
        pltpu.VMEM((C, F), jnp.float32),
        pltpu.VMEM((C, SD), jnp.float32),
        pltpu.VMEM((C, WD), jnp.float32),
        pltpu.SemaphoreType.DMA,
    ],
)
def _emb_kernel(feat_hbm, sids_hbm, wids_hbm, stab_hbm, wtab_hbm, out_hbm,
                widx_v, sidx_v, feat_v, srows_v, wrows_v, sem):
    wid = lax.axis_index("s") * NUM_CORES + lax.axis_index("c")
    base0 = wid * TOK_W

    def body(it, carry):
        base = base0 + it * C
        pltpu.sync_copy(wids_hbm.at[pl.ds(base, C)], widx_v)
        pltpu.sync_copy(sids_hbm.at[pl.ds(base, C)], sidx_v)
        g_w = pltpu.async_copy(wtab_hbm.at[widx_v], wrows_v, sem)
        g_s = pltpu.async_copy(stab_hbm.at[sidx_v], srows_v, sem)
        g_f = pltpu.async_copy(feat_hbm.at[pl.ds(base, C)], feat_v, sem)
        g_w.wait()
        g_s.wait()
        g_f.wait()
        o_f = pltpu.async_copy(feat_v, out_hbm.at[pl.ds(base, C), pl.ds(0, F)], sem)
        o_s = pltpu.async_copy(srows_v, out_hbm.at[pl.ds(base, C), pl.ds(F, SD)], sem)
        o_w = pltpu.async_copy(wrows_v, out_hbm.at[pl.ds(base, C), pl.ds(F + SD, WD)], sem)
        o_f.wait()
        o_s.wait()
        o_w.wait()
        return carry

    lax.fori_loop(0, ITERS, body, 0)


def kernel(feature_tensor, shape_ids, word_ids, shape_table, word_table):
    feat = feature_tensor.reshape(N, F)
    sids = shape_ids.reshape(N).astype(jnp.int32)
    wids = word_ids.reshape(N).astype(jnp.int32)
    out = _emb_kernel(feat, sids, wids, shape_table, word_table)
    return out.reshape(B, S, OUT_D)


# speedup vs baseline: 1.9419x; 1.0996x over previous
import functools

import jax
import jax.numpy as jnp
from jax import lax
from jax.experimental import pallas as pl
from jax.experimental.pallas import tpu as pltpu
from jax.experimental.pallas import tpu_sc as plsc

B, S, F = 1024, 200, 64
SD, WD = 32, 32
OUT_D = F + SD + WD
N = B * S
NUM_CORES = 2
NUM_SUBCORES = 16
NW = NUM_CORES * NUM_SUBCORES
TOK_W = N // NW
C = 128
ITERS = TOK_W // C

_mesh = plsc.VectorSubcoreMesh(core_axis_name="c", subcore_axis_name="s")


@functools.partial(
    pl.kernel,
    mesh=_mesh,
    compiler_params=pltpu.CompilerParams(use_tc_tiling_on_sc=False),
    out_type=jax.ShapeDtypeStruct((N, OUT_D), jnp.float32),
    scratch_types=[
        pltpu.VMEM((TOK_W,), jnp.int32),
        pltpu.VMEM((TOK_W,), jnp.int32),
        pltpu.VMEM((2, C, F), jnp.float32),
        pltpu.VMEM((2, C, SD), jnp.float32),
        pltpu.VMEM((2, C, WD), jnp.float32),
        pltpu.SemaphoreType.DMA((2,)),
        pltpu.SemaphoreType.DMA((2,)),
    ],
)
def _emb_kernel(feat_hbm, sids_hbm, wids_hbm, stab_hbm, wtab_hbm, out_hbm,
                widx_v, sidx_v, feat_v, srows_v, wrows_v, in_sem, out_sem):
    wid = lax.axis_index("s") * NUM_CORES + lax.axis_index("c")
    base0 = wid * TOK_W
    pltpu.sync_copy(wids_hbm.at[pl.ds(base0, TOK_W)], widx_v)
    pltpu.sync_copy(sids_hbm.at[pl.ds(base0, TOK_W)], sidx_v)

    def start_in(c, p):
        base = base0 + c * C
        pltpu.async_copy(feat_hbm.at[pl.ds(base, C)], feat_v.at[p], in_sem.at[p])
        pltpu.async_copy(
            stab_hbm.at[sidx_v.at[pl.ds(c * C, C)]], srows_v.at[p], in_sem.at[p])
        pltpu.async_copy(
            wtab_hbm.at[widx_v.at[pl.ds(c * C, C)]], wrows_v.at[p], in_sem.at[p])

    def wait_in(p):
        pltpu.make_async_copy(
            feat_hbm.at[pl.ds(0, C)], feat_v.at[p], in_sem.at[p]).wait()
        pltpu.make_async_copy(
            stab_hbm.at[pl.ds(0, C)], srows_v.at[p], in_sem.at[p]).wait()
        pltpu.make_async_copy(
            wtab_hbm.at[pl.ds(0, C)], wrows_v.at[p], in_sem.at[p]).wait()

    def start_out(c, p):
        base = base0 + c * C
        pltpu.async_copy(
            feat_v.at[p], out_hbm.at[pl.ds(base, C), pl.ds(0, F)], out_sem.at[p])
        pltpu.async_copy(
            srows_v.at[p], out_hbm.at[pl.ds(base, C), pl.ds(F, SD)], out_sem.at[p])
        pltpu.async_copy(
            wrows_v.at[p], out_hbm.at[pl.ds(base, C), pl.ds(F + SD, WD)],
            out_sem.at[p])

    def wait_out(c, p):
        base = base0 + c * C
        pltpu.make_async_copy(
            feat_v.at[p], out_hbm.at[pl.ds(base, C), pl.ds(0, F)],
            out_sem.at[p]).wait()
        pltpu.make_async_copy(
            srows_v.at[p], out_hbm.at[pl.ds(base, C), pl.ds(F, SD)],
            out_sem.at[p]).wait()
        pltpu.make_async_copy(
            wrows_v.at[p], out_hbm.at[pl.ds(base, C), pl.ds(F + SD, WD)],
            out_sem.at[p]).wait()

    start_in(0, 0)

    def body(it, carry):
        p = lax.rem(it, 2)
        q = 1 - p

        @pl.when(it >= 1)
        def _():
            wait_out(it - 1, q)

        @pl.when(it < ITERS - 1)
        def _():
            start_in(it + 1, q)

        wait_in(p)
        start_out(it, p)
        return carry

    lax.fori_loop(0, ITERS, body, 0)
    wait_out(ITERS - 1, lax.rem(ITERS - 1, 2))


def kernel(feature_tensor, shape_ids, word_ids, shape_table, word_table):
    feat = feature_tensor.reshape(N, F)
    sids = shape_ids.reshape(N).astype(jnp.int32)
    wids = word_ids.reshape(N).astype(jnp.int32)
    out = _emb_kernel(feat, sids, wids, shape_table, word_table)
    return out.reshape(B, S, OUT_D)
